# baseline (speedup 1.0000x reference)
import functools

import jax
import jax.numpy as jnp
from jax import lax
from jax.experimental import pallas as pl
from jax.experimental.pallas import tpu as pltpu
from jax.experimental.pallas import tpu_sc as plsc

N = 320000
D = 128
H = 16
NSEG = 10000

NC = 2
NS = 16
NW = NC * NS
IR = N // D
IR_PAD = 2512

IR_TC = 1000
IR_SC = IR - IR_TC
IR_BASE = IR_SC // NW
IR_EXTRA = IR_SC % NW
MAX_IRPT = IR_BASE + 1
NG = IR_BASE
IDXBUF = 64
ZU = (NSEG + D - 1) // D
ZTAIL = NSEG - (ZU - 1) * D

BI = 8
NB = IR_TC // BI
ACC_PAD = 81 * D


def _sc_segment_sum(x, batch2d):
    mesh = plsc.VectorSubcoreMesh(core_axis_name="c", subcore_axis_name="s")

    @functools.partial(
        pl.kernel,
        mesh=mesh,
        out_type=jax.ShapeDtypeStruct((NC, NSEG, D), jnp.float32),
        scratch_types=[
            pltpu.VMEM((2, D, D), jnp.float32),
            pltpu.VMEM((IDXBUF, D), jnp.int32),
            pltpu.VMEM_SHARED((NSEG, D), jnp.float32),
            pltpu.SemaphoreType.DMA,
            pltpu.SemaphoreType.DMA,
            pltpu.SemaphoreType.DMA,
            pltpu.SemaphoreType.DMA,
        ],
    )
    def seg_kernel(x_hbm, b_hbm, out_hbm, rows_v, idx_v, acc_sh,
                   sem_g0, sem_g1, sem_s0, sem_s1):
        c = lax.axis_index("c")
        s = lax.axis_index("s")
        w = c * NS + s

        nck = IR_BASE + (w < IR_EXTRA).astype(jnp.int32)
        a = IR_TC + IR_BASE * w + jnp.minimum(w, IR_EXTRA)
        sa = (a // 8) * 8
        o = a - sa

        zeros16 = jnp.zeros((16,), jnp.float32)

        def zbody(i, carry):
            r = i // (D // 16)
            q = i % (D // 16)
            rows_v[0, r, pl.ds(q * 16, 16)] = zeros16
            return carry

        lax.fori_loop(0, D * (D // 16), zbody, 0)
        for k in range(5):
            u = s + NS * k

            @pl.when(u < ZU - 1)
            def _():
                pltpu.sync_copy(
                    rows_v.at[0, pl.ds(0, D)], acc_sh.at[pl.ds(u * D, D)]
                )

            @pl.when(u == ZU - 1)
            def _():
                pltpu.sync_copy(
                    rows_v.at[0, pl.ds(0, ZTAIL)],
                    acc_sh.at[pl.ds((ZU - 1) * D, ZTAIL)],
                )
        plsc.subcore_barrier()

        pltpu.sync_copy(b_hbm.at[pl.ds(sa, IDXBUF)], idx_v)

        gsems = (sem_g0, sem_g1)
        ssems = (sem_s0, sem_s1)

        def gather(k, buf):
            return pltpu.make_async_copy(
                x_hbm.at[pl.ds((a + k) * D, D)], rows_v.at[buf], gsems[buf]
            )

        def scatter(k, buf):
            return pltpu.make_async_copy(
                rows_v.at[buf], acc_sh.at[idx_v.at[o + k]], ssems[buf]
            )

        gather(0, 0).start()
        for k in range(NG):
            b = k % 2
            gather(k, b).wait()
            if k + 1 < NG:
                if k >= 1:
                    scatter(k - 1, 1 - b).wait()
                gather(k + 1, 1 - b).start()
            scatter(k, b).start(add=True)
        scatter(NG - 2, (NG - 2) % 2).wait()
        scatter(NG - 1, (NG - 1) % 2).wait()

        @pl.when(nck == MAX_IRPT)
        def _():
            pltpu.sync_copy(
                x_hbm.at[pl.ds((a + IR_BASE) * D, D)],
                rows_v.at[0, pl.ds(0, D)],
            )
            pltpu.sync_copy(
                rows_v.at[0, pl.ds(0, D)],
                acc_sh.at[idx_v.at[o + IR_BASE]],
                add=True,
            )
        plsc.subcore_barrier()

        for k in range(5):
            u = s + NS * k

            @pl.when(u < ZU - 1)
            def _():
                pltpu.sync_copy(
                    acc_sh.at[pl.ds(u * D, D)],
                    out_hbm.at[c, pl.ds(u * D, D)],
                )

            @pl.when(u == ZU - 1)
            def _():
                pltpu.sync_copy(
                    acc_sh.at[pl.ds((ZU - 1) * D, ZTAIL)],
                    out_hbm.at[c, pl.ds((ZU - 1) * D, ZTAIL)],
                )

    return seg_kernel(x, batch2d)


def _tc_segment_sum(x, batch_col):
    def tc_kernel(x_ref, idx_ref, o_ref, acc_ref):
        i = pl.program_id(0)

        @pl.when(i == 0)
        def _():
            acc_ref[...] = jnp.zeros((ACC_PAD, D), jnp.float32)

        idx = idx_ref[...]
        base = idx[0, 0]
        pmax = (idx[BI * D - 1, 0] - base) // D + 1
        xb = x_ref[...]
        colv = lax.broadcasted_iota(jnp.int32, (1, D), 1)

        def body(p, carry):
            off = idx - base - p * D
            oh = (off == colv).astype(jnp.float32)
            part = lax.dot_general(
                oh, xb, (((0,), (0,)), ((), ())),
                preferred_element_type=jnp.float32,
            )
            ss = base + p * D
            acc_ref[pl.ds(ss, D), :] += part
            return carry

        lax.fori_loop(0, pmax, body, 0)

        @pl.when(i == pl.num_programs(0) - 1)
        def _():
            o_ref[...] = acc_ref[pl.ds(0, NSEG), :]

    return pl.pallas_call(
        tc_kernel,
        grid=(NB,),
        in_specs=[
            pl.BlockSpec((BI * D, D), lambda i: (i, 0)),
            pl.BlockSpec((BI * D, 1), lambda i: (i, 0)),
        ],
        out_specs=pl.BlockSpec((NSEG, D), lambda i: (0, 0)),
        out_shape=jax.ShapeDtypeStruct((NSEG, D), jnp.float32),
        scratch_shapes=[pltpu.VMEM((ACC_PAD, D), jnp.float32)],
    )(x, batch_col)


def _mlp(partials, tc_pooled, W1, b1, W2, b2):
    def mlp_kernel(p_ref, t_ref, w1_ref, b1_ref, w2_ref, b2_ref, o_ref):
        pooled = p_ref[0] + p_ref[1] + t_ref[...]
        h = jnp.dot(pooled, w1_ref[...], preferred_element_type=jnp.float32)
        h = jnp.maximum(h + b1_ref[...], 0.0)
        o = jnp.dot(h, w2_ref[...], preferred_element_type=jnp.float32)
        o_ref[...] = o + b2_ref[...]

    return pl.pallas_call(
        mlp_kernel,
        out_shape=jax.ShapeDtypeStruct((NSEG, 1), jnp.float32),
    )(partials, tc_pooled, W1, b1.reshape(1, H), W2, b2.reshape(1, 1))


def kernel(x, batch, W1, b1, W2, b2):
    bi = batch.astype(jnp.int32)
    batch2d = jnp.pad(bi, (0, IR_PAD * D - N)).reshape(IR_PAD, D)
    partials = _sc_segment_sum(x, batch2d)
    tc_pooled = _tc_segment_sum(x, bi[: IR_TC * D].reshape(IR_TC * D, 1))
    out = _mlp(partials, tc_pooled, W1, b1, W2, b2)
    return out.reshape(NSEG)


# speedup vs baseline: 1.0012x; 1.0012x over previous
import functools

import jax
import jax.numpy as jnp
from jax import lax
from jax.experimental import pallas as pl
from jax.experimental.pallas import tpu as pltpu
from jax.experimental.pallas import tpu_sc as plsc

N = 320000
D = 128
H = 16
NSEG = 10000

NC = 2
NS = 16
NW = NC * NS
IR = N // D
IR_PAD = 2512

IR_TC = 1000
IR_SC = IR - IR_TC
IR_BASE = IR_SC // NW
IR_EXTRA = IR_SC % NW
MAX_IRPT = IR_BASE + 1
NG = IR_BASE
IDXBUF = 64
ZU = (NSEG + D - 1) // D
ZTAIL = NSEG - (ZU - 1) * D

BI = 8
NB = IR_TC // BI
ACC_PAD = 81 * D


def _sc_segment_sum(x, batch2d):
    mesh = plsc.VectorSubcoreMesh(core_axis_name="c", subcore_axis_name="s")

    @functools.partial(
        pl.kernel,
        mesh=mesh,
        out_type=jax.ShapeDtypeStruct((NC, NSEG, D), jnp.float32),
        scratch_types=[
            pltpu.VMEM((2, D, D), jnp.float32),
            pltpu.VMEM((IDXBUF, D), jnp.int32),
            pltpu.VMEM_SHARED((NSEG, D), jnp.float32),
            pltpu.SemaphoreType.DMA,
            pltpu.SemaphoreType.DMA,
            pltpu.SemaphoreType.DMA,
            pltpu.SemaphoreType.DMA,
        ],
    )
    def seg_kernel(x_hbm, b_hbm, out_hbm, rows_v, idx_v, acc_sh,
                   sem_g0, sem_g1, sem_s0, sem_s1):
        c = lax.axis_index("c")
        s = lax.axis_index("s")
        w = c * NS + s

        nck = IR_BASE + (w < IR_EXTRA).astype(jnp.int32)
        a = IR_TC + IR_BASE * w + jnp.minimum(w, IR_EXTRA)
        sa = (a // 8) * 8
        o = a - sa

        zeros16 = jnp.zeros((16,), jnp.float32)

        def zbody(i, carry):
            r = i // (D // 16)
            q = i % (D // 16)
            rows_v[0, r, pl.ds(q * 16, 16)] = zeros16
            return carry

        lax.fori_loop(0, D * (D // 16), zbody, 0)
        for k in range(5):
            u = s + NS * k

            @pl.when(u < ZU - 1)
            def _():
                pltpu.sync_copy(
                    rows_v.at[0, pl.ds(0, D)], acc_sh.at[pl.ds(u * D, D)]
                )

            @pl.when(u == ZU - 1)
            def _():
                pltpu.sync_copy(
                    rows_v.at[0, pl.ds(0, ZTAIL)],
                    acc_sh.at[pl.ds((ZU - 1) * D, ZTAIL)],
                )
        plsc.subcore_barrier()

        pltpu.sync_copy(b_hbm.at[pl.ds(sa, IDXBUF)], idx_v)

        gsems = (sem_g0, sem_g1)
        ssems = (sem_s0, sem_s1)

        def gather(k, buf):
            return pltpu.make_async_copy(
                x_hbm.at[pl.ds((a + k) * D, D)], rows_v.at[buf], gsems[buf]
            )

        def scatter(k, buf):
            return pltpu.make_async_copy(
                rows_v.at[buf], acc_sh.at[idx_v.at[o + k]], ssems[buf]
            )

        gather(0, 0).start()
        for k in range(NG):
            b = k % 2
            gather(k, b).wait()
            if k + 1 < NG:
                if k >= 1:
                    scatter(k - 1, 1 - b).wait()
                gather(k + 1, 1 - b).start()
            scatter(k, b).start(add=True)
        scatter(NG - 2, (NG - 2) % 2).wait()
        scatter(NG - 1, (NG - 1) % 2).wait()

        @pl.when(nck == MAX_IRPT)
        def _():
            pltpu.sync_copy(
                x_hbm.at[pl.ds((a + IR_BASE) * D, D)],
                rows_v.at[0, pl.ds(0, D)],
            )
            pltpu.sync_copy(
                rows_v.at[0, pl.ds(0, D)],
                acc_sh.at[idx_v.at[o + IR_BASE]],
                add=True,
            )
        plsc.subcore_barrier()

        for k in range(5):
            u = s + NS * k

            @pl.when(u < ZU - 1)
            def _():
                pltpu.sync_copy(
                    acc_sh.at[pl.ds(u * D, D)],
                    out_hbm.at[c, pl.ds(u * D, D)],
                )

            @pl.when(u == ZU - 1)
            def _():
                pltpu.sync_copy(
                    acc_sh.at[pl.ds((ZU - 1) * D, ZTAIL)],
                    out_hbm.at[c, pl.ds((ZU - 1) * D, ZTAIL)],
                )

    return seg_kernel(x, batch2d)


def _tc_segment_sum(x, batch_col):
    def tc_kernel(x_ref, idx_ref, o_ref, acc_ref):
        i = pl.program_id(0)

        @pl.when(i == 0)
        def _():
            acc_ref[...] = jnp.zeros((ACC_PAD, D), jnp.float32)

        idx = idx_ref[...]
        base = idx[0, 0]
        pmax = (idx[BI * D - 1, 0] - base) // D + 1
        xb = x_ref[...]
        colv = lax.broadcasted_iota(jnp.int32, (1, D), 1)

        def body(p, carry):
            off = idx - base - p * D
            oh = (off == colv).astype(jnp.float32)
            part = lax.dot_general(
                oh, xb, (((0,), (0,)), ((), ())),
                preferred_element_type=jnp.float32,
            )
            ss = base + p * D
            acc_ref[pl.ds(ss, D), :] += part
            return carry

        lax.fori_loop(0, pmax, body, 0)

        @pl.when(i == pl.num_programs(0) - 1)
        def _():
            o_ref[...] = acc_ref[pl.ds(0, NSEG), :]

    return pl.pallas_call(
        tc_kernel,
        grid=(NB,),
        in_specs=[
            pl.BlockSpec((BI * D, D), lambda i: (i, 0)),
            pl.BlockSpec((BI * D, 1), lambda i: (i, 0)),
        ],
        out_specs=pl.BlockSpec((NSEG, D), lambda i: (0, 0)),
        out_shape=jax.ShapeDtypeStruct((NSEG, D), jnp.float32),
        scratch_shapes=[pltpu.VMEM((ACC_PAD, D), jnp.float32)],
    )(x, batch_col)


def _mlp(partials, tc_pooled, W1, b1, W2, b2):
    def mlp_kernel(p_ref, t_ref, w1_ref, b1_ref, w2_ref, b2_ref, o_ref):
        pooled = p_ref[0] + p_ref[1] + t_ref[...]
        h = jnp.dot(pooled, w1_ref[...], preferred_element_type=jnp.float32)
        h = jnp.maximum(h + b1_ref[...], 0.0)
        o = jnp.dot(h, w2_ref[...], preferred_element_type=jnp.float32)
        o_ref[...] = o + b2_ref[...]

    return pl.pallas_call(
        mlp_kernel,
        out_shape=jax.ShapeDtypeStruct((NSEG, 1), jnp.float32),
    )(partials, tc_pooled, W1, b1.reshape(1, H), W2, b2.reshape(1, 1))


def kernel(x, batch, W1, b1, W2, b2):
    bi = batch.astype(jnp.int32)
    batch2d = jnp.pad(bi, (0, IR_PAD * D - N)).reshape(IR_PAD, D)
    tc_pooled = _tc_segment_sum(x, bi[: IR_TC * D].reshape(IR_TC * D, 1))
    partials = _sc_segment_sum(x, batch2d)
    out = _mlp(partials, tc_pooled, W1, b1, W2, b2)
    return out.reshape(NSEG)


# speedup vs baseline: 1.2405x; 1.2390x over previous
import functools

import jax
import jax.numpy as jnp
from jax import lax
from jax.experimental import pallas as pl
from jax.experimental.pallas import tpu as pltpu
from jax.experimental.pallas import tpu_sc as plsc

N = 320000
D = 128
H = 16
NSEG = 10000

NC = 2
NS = 16
NW = NC * NS
IR = N // D
IR_PAD = 2504
IR_BASE = IR // NW
IR_EXTRA = IR % NW
MAX_IRPT = IR_BASE + 1
IDXBUF = 88
G = 1
NG = IR_BASE // G
ZU = (NSEG + D - 1) // D
ZTAIL = NSEG - (ZU - 1) * D


def _sc_segment_sum(x, batch2d):
    mesh = plsc.VectorSubcoreMesh(core_axis_name="c", subcore_axis_name="s")

    @functools.partial(
        pl.kernel,
        mesh=mesh,
        out_type=jax.ShapeDtypeStruct((NC, NSEG, D), jnp.float32),
        scratch_types=[
            pltpu.VMEM((2, G * D, D), jnp.float32),
            pltpu.VMEM((IDXBUF, D), jnp.int32),
            pltpu.VMEM_SHARED((NSEG, D), jnp.float32),
            pltpu.SemaphoreType.DMA,
            pltpu.SemaphoreType.DMA,
            pltpu.SemaphoreType.DMA,
            pltpu.SemaphoreType.DMA,
        ],
    )
    def seg_kernel(x_hbm, b_hbm, out_hbm, rows_v, idx_v, acc_sh,
                   sem_g0, sem_g1, sem_s0, sem_s1):
        c = lax.axis_index("c")
        s = lax.axis_index("s")
        w = c * NS + s

        nck = IR_BASE + (w < IR_EXTRA).astype(jnp.int32)
        a = IR_BASE * w + jnp.minimum(w, IR_EXTRA)
        sa = (a // 8) * 8
        o = a - sa

        gsems = (sem_g0, sem_g1)
        ssems = (sem_s0, sem_s1)

        def gather(k, buf):
            return pltpu.make_async_copy(
                x_hbm.at[pl.ds((a + k) * D, D)], rows_v.at[buf], gsems[buf]
            )

        def scatter(k, buf):
            return pltpu.make_async_copy(
                rows_v.at[buf], acc_sh.at[idx_v.at[o + k]], ssems[buf]
            )

        idx_fetch = pltpu.make_async_copy(
            b_hbm.at[pl.ds(sa, IDXBUF)], idx_v, sem_s1
        )
        idx_fetch.start()

        zeros16 = jnp.zeros((16,), jnp.float32)

        def zbody(i, carry):
            r = i // (D // 16)
            q = i % (D // 16)
            rows_v[0, r, pl.ds(q * 16, 16)] = zeros16
            return carry

        lax.fori_loop(0, D * (D // 16), zbody, 0)
        zcopies = []
        for k in range(5):
            u = s + NS * k

            @pl.when(u < ZU - 1)
            def _():
                pltpu.async_copy(
                    rows_v.at[0, pl.ds(0, D)],
                    acc_sh.at[pl.ds(u * D, D)],
                    sem_s0,
                )

            @pl.when(u == ZU - 1)
            def _():
                pltpu.async_copy(
                    rows_v.at[0, pl.ds(0, ZTAIL)],
                    acc_sh.at[pl.ds((ZU - 1) * D, ZTAIL)],
                    sem_s0,
                )
        gather(1, 1).start()
        for k in range(5):
            u = s + NS * k

            @pl.when(u < ZU - 1)
            def _():
                pltpu.make_async_copy(
                    rows_v.at[0, pl.ds(0, D)],
                    acc_sh.at[pl.ds(u * D, D)],
                    sem_s0,
                ).wait()

            @pl.when(u == ZU - 1)
            def _():
                pltpu.make_async_copy(
                    rows_v.at[0, pl.ds(0, ZTAIL)],
                    acc_sh.at[pl.ds((ZU - 1) * D, ZTAIL)],
                    sem_s0,
                ).wait()
        gather(0, 0).start()
        idx_fetch.wait()
        plsc.subcore_barrier()

        for k in range(NG):
            b = k % 2
            gather(k, b).wait()
            if k + 1 < NG and k >= 1:
                scatter(k - 1, 1 - b).wait()
                gather(k + 1, 1 - b).start()
            scatter(k, b).start(add=True)
        scatter(NG - 2, (NG - 2) % 2).wait()
        scatter(NG - 1, (NG - 1) % 2).wait()

        @pl.when(nck == MAX_IRPT)
        def _():
            pltpu.sync_copy(
                x_hbm.at[pl.ds((a + IR_BASE) * D, D)],
                rows_v.at[0, pl.ds(0, D)],
            )
            pltpu.sync_copy(
                rows_v.at[0, pl.ds(0, D)],
                acc_sh.at[idx_v.at[o + IR_BASE]],
                add=True,
            )
        plsc.subcore_barrier()

        for wait in (False, True):
            for k in range(5):
                u = s + NS * k

                @pl.when(u < ZU - 1)
                def _():
                    cp = pltpu.make_async_copy(
                        acc_sh.at[pl.ds(u * D, D)],
                        out_hbm.at[c, pl.ds(u * D, D)],
                        sem_g0,
                    )
                    cp.wait() if wait else cp.start()

                @pl.when(u == ZU - 1)
                def _():
                    cp = pltpu.make_async_copy(
                        acc_sh.at[pl.ds((ZU - 1) * D, ZTAIL)],
                        out_hbm.at[c, pl.ds((ZU - 1) * D, ZTAIL)],
                        sem_g0,
                    )
                    cp.wait() if wait else cp.start()

    return seg_kernel(x, batch2d)


def _mlp(partials, W1, b1, W2, b2):
    def mlp_kernel(p_ref, w1_ref, b1_ref, w2_ref, b2_ref, o_ref):
        pooled = p_ref[0] + p_ref[1]
        h = jnp.dot(pooled, w1_ref[...], preferred_element_type=jnp.float32)
        h = jnp.maximum(h + b1_ref[...], 0.0)
        o = jnp.dot(h, w2_ref[...], preferred_element_type=jnp.float32)
        o_ref[...] = o + b2_ref[...]

    return pl.pallas_call(
        mlp_kernel,
        out_shape=jax.ShapeDtypeStruct((NSEG, 1), jnp.float32),
    )(partials, W1, b1.reshape(1, H), W2, b2.reshape(1, 1))


def kernel(x, batch, W1, b1, W2, b2):
    bi = batch.astype(jnp.int32)
    bi = jnp.pad(bi, (0, IR_PAD * D - N))
    batch2d = bi.reshape(IR_PAD, D)
    partials = _sc_segment_sum(x, batch2d)
    out = _mlp(partials, W1, b1, W2, b2)
    return out.reshape(NSEG)
